# baseline (speedup 1.0000x reference)
import functools

import jax
import jax.numpy as jnp
from jax import lax
from jax.experimental import pallas as pl
from jax.experimental.pallas import tpu as pltpu
from jax.experimental.pallas import tpu_sc as plsc

NC = 2
NS = 16
NW = NC * NS
CHUNK = 128


def _round_up(v, m):
    return (v + m - 1) // m * m


@functools.partial(jax.jit, static_argnames=("n", "np_", "npt", "cpwf", "cpws"))
def _gcn_pallas(x, sa, sb, da, db, W2, b2, Wc, bc, n, np_, npt, cpwf, cpws):
    _, D = x.shape
    N = n
    H = W2.shape[1]
    C = Wc.shape[1]
    NP = np_
    NPT = npt
    CPWF = cpwf
    CPWS = cpws

    mesh = plsc.VectorSubcoreMesh(core_axis_name="c", subcore_axis_name="s")

    @functools.partial(
        pl.kernel,
        out_type=jax.ShapeDtypeStruct((NC, NP), jnp.float32),
        mesh=mesh,
        scratch_types=[
            pltpu.VMEM((CPWF, CHUNK), jnp.int32),
            pltpu.VMEM((CHUNK,), jnp.float32),
            pltpu.VMEM((NPT,), jnp.float32),
            pltpu.VMEM_SHARED((NP,), jnp.float32),
        ],
    )
    def deg_kernel(da_hbm, db_hbm, out_hbm, idxs, ones_v, stage, deg_sh):
        c = lax.axis_index("c")
        s = lax.axis_index("s")
        nw = jnp.where(c == 0, CPWF, CPWS)

        @pl.when(c == 0)
        def _():
            pltpu.sync_copy(da_hbm.at[s], idxs)

        @pl.when(c != 0)
        def _():
            pltpu.sync_copy(db_hbm.at[s], idxs.at[pl.ds(0, CPWS)])
        for i in range(CHUNK // 16):
            ones_v[pl.ds(i * 16, 16)] = jnp.full((16,), 1.0, jnp.float32)
        for i in range(NPT // 16):
            stage[pl.ds(i * 16, 16)] = jnp.zeros((16,), jnp.float32)
        pltpu.sync_copy(stage, deg_sh.at[pl.ds(s * NPT, NPT)])
        plsc.subcore_barrier()

        @pl.loop(0, nw)
        def _(j):
            pltpu.sync_copy(ones_v, deg_sh.at[idxs.at[j]], add=True)

        plsc.subcore_barrier()
        pltpu.sync_copy(deg_sh.at[pl.ds(s * NPT, NPT)], stage)
        pltpu.sync_copy(stage, out_hbm.at[c, pl.ds(s * NPT, NPT)])

    degp = deg_kernel(da, db)
    d0 = degp[0].reshape(NP, 1)
    d1 = degp[1].reshape(NP, 1)

    GB = 8
    RB = NP // GB
    HH = H // 2
    W2s = W2.reshape(D, 2, HH).transpose(1, 0, 2)

    def tc1_body(x_ref, w_ref, d0_ref, d1_ref, g_ref):
        dinv = lax.rsqrt(d0_ref[...] + d1_ref[...] + 1.0)
        xb = x_ref[...]
        g_ref[0] = jnp.dot(xb, w_ref[0],
                           preferred_element_type=jnp.float32) * dinv
        g_ref[1] = jnp.dot(xb, w_ref[1],
                           preferred_element_type=jnp.float32) * dinv

    g2 = pl.pallas_call(
        tc1_body,
        grid=(GB,),
        in_specs=[
            pl.BlockSpec((RB, D), lambda i: (i, 0)),
            pl.BlockSpec((2, D, HH), lambda i: (0, 0, 0)),
            pl.BlockSpec((RB, 1), lambda i: (i, 0)),
            pl.BlockSpec((RB, 1), lambda i: (i, 0)),
        ],
        out_specs=pl.BlockSpec((2, RB, HH), lambda i: (0, i, 0)),
        out_shape=jax.ShapeDtypeStruct((2, NP, HH), jnp.float32),
    )(x, W2s, d0, d1)

    @functools.partial(
        pl.kernel,
        out_type=jax.ShapeDtypeStruct((2, NC, NP, HH), jnp.float32),
        mesh=mesh,
        compiler_params=pltpu.CompilerParams(use_tc_tiling_on_sc=False),
        scratch_types=[
            pltpu.VMEM((CPWF, CHUNK), jnp.int32),
            pltpu.VMEM((CPWF, CHUNK), jnp.int32),
            pltpu.VMEM((CHUNK, HH), jnp.float32),
            pltpu.VMEM((CHUNK, HH), jnp.float32),
            pltpu.VMEM_SHARED((NP, HH), jnp.float32),
            pltpu.SemaphoreType.DMA,
            pltpu.SemaphoreType.DMA,
        ],
    )
    def agg_kernel(g2_hbm, sa_hbm, sb_hbm, da_hbm, db_hbm, o_hbm,
                   sidxs, didxs, r0, r1, agg_sh, sem0, sem1):
        c = lax.axis_index("c")
        s = lax.axis_index("s")
        nw = jnp.where(c == 0, CPWF, CPWS)

        @pl.when(c == 0)
        def _():
            pltpu.sync_copy(sa_hbm.at[s], sidxs)
            pltpu.sync_copy(da_hbm.at[s], didxs)

        @pl.when(c != 0)
        def _():
            pltpu.sync_copy(sb_hbm.at[s], sidxs.at[pl.ds(0, CPWS)])
            pltpu.sync_copy(db_hbm.at[s], didxs.at[pl.ds(0, CPWS)])

        for p in range(2):
            g_hbm = g2_hbm.at[p]
            out_hbm = o_hbm.at[p]
            @pl.loop(0, NPT // CHUNK)
            def _(i):
                base = s * NPT + i * CHUNK
                pltpu.sync_copy(g_hbm.at[pl.ds(base, CHUNK)], r0)
                pltpu.sync_copy(r0, agg_sh.at[pl.ds(base, CHUNK)])

            plsc.subcore_barrier()

            pltpu.async_copy(g_hbm.at[sidxs.at[0]], r0, sem0)

            @pl.loop(0, nw, step=2)
            def _(j):
                pltpu.async_copy(g_hbm.at[sidxs.at[j + 1]], r1, sem1)
                pltpu.make_async_copy(g_hbm.at[sidxs.at[j]], r0, sem0).wait()
                pltpu.sync_copy(r0, agg_sh.at[didxs.at[j]], add=True)

                @pl.when(j + 2 < nw)
                def _():
                    pltpu.async_copy(g_hbm.at[sidxs.at[j + 2]], r0, sem0)

                pltpu.make_async_copy(g_hbm.at[sidxs.at[j + 1]], r1,
                                      sem1).wait()
                pltpu.sync_copy(r1, agg_sh.at[didxs.at[j + 1]], add=True)

            plsc.subcore_barrier()

            @pl.loop(0, NPT // CHUNK)
            def _(i):
                base = s * NPT + i * CHUNK
                pltpu.sync_copy(agg_sh.at[pl.ds(base, CHUNK)], r0)
                pltpu.sync_copy(r0, out_hbm.at[c, pl.ds(base, CHUNK)])

            plsc.subcore_barrier()

    o4 = agg_kernel(g2, sa, sb, da, db)

    def tc2_body(a00, a01, a10, a11, ga_ref, gb_ref, d0_ref, d1_ref,
                 b2l, b2r, wcl, wcr, bc_ref, o_ref):
        dinv = lax.rsqrt(d0_ref[...] + d1_ref[...] + 1.0)
        pre_l = (a00[0, 0] + a01[0, 0] - ga_ref[0]) * dinv + b2l[...]
        pre_r = (a10[0, 0] + a11[0, 0] - gb_ref[0]) * dinv + b2r[...]
        pre_l = jnp.maximum(pre_l, 0.0)
        pre_r = jnp.maximum(pre_r, 0.0)
        acc = jnp.dot(pre_l, wcl[...], preferred_element_type=jnp.float32)
        acc += jnp.dot(pre_r, wcr[...], preferred_element_type=jnp.float32)
        o_ref[...] = acc + bc_ref[...]

    GB2 = 10
    RB2 = N // GB2

    def _ospec(p, cc):
        return pl.BlockSpec((1, 1, RB2, HH),
                            lambda i, p=p, cc=cc: (p, cc, i, 0))

    def _gspec(p):
        return pl.BlockSpec((1, RB2, HH), lambda i, p=p: (p, i, 0))

    out = pl.pallas_call(
        tc2_body,
        grid=(GB2,),
        in_specs=[
            _ospec(0, 0), _ospec(0, 1), _ospec(1, 0), _ospec(1, 1),
            _gspec(0), _gspec(1),
            pl.BlockSpec((RB2, 1), lambda i: (i, 0)),
            pl.BlockSpec((RB2, 1), lambda i: (i, 0)),
            pl.BlockSpec((1, HH), lambda i: (0, 0)),
            pl.BlockSpec((1, HH), lambda i: (0, 0)),
            pl.BlockSpec((HH, C), lambda i: (0, 0)),
            pl.BlockSpec((HH, C), lambda i: (0, 0)),
            pl.BlockSpec((1, C), lambda i: (0, 0)),
        ],
        out_specs=pl.BlockSpec((RB2, C), lambda i: (i, 0)),
        out_shape=jax.ShapeDtypeStruct((N, C), jnp.float32),
    )(o4, o4, o4, o4, g2, g2, d0, d1,
      b2[:HH].reshape(1, HH), b2[HH:].reshape(1, HH),
      Wc[:HH], Wc[HH:], bc.reshape(1, C))

    return out


FAST_FRAC = 0.72


def _split_counts(E):
    tch = -(-E // CHUNK)
    cpwf = _round_up(-(-int(tch * FAST_FRAC) // NS), 2)
    rem = max(E - NS * cpwf * CHUNK, 0)
    cpws = max(_round_up(-(-rem // (NS * CHUNK)), 2), 2)
    return cpwf, cpws


def _edges2(v, cpwf, cpws, padval):
    E = v.shape[0]
    ea = NS * cpwf * CHUNK
    cap = NS * (cpwf + cpws) * CHUNK
    vp = jnp.concatenate([v, jnp.full((cap - E,), padval, jnp.int32)])
    return (vp[:ea].reshape(NS, cpwf, CHUNK),
            vp[ea:].reshape(NS, cpws, CHUNK))


def kernel(x, edge_index, W2, b2, Wc, bc):
    N, D = x.shape
    H = W2.shape[1]
    C = Wc.shape[1]
    E = edge_index.shape[1]

    NP = _round_up(N + 1, NS * CHUNK)
    NPT = NP // NS
    CPWF, CPWS = _split_counts(E)

    ei = edge_index.astype(jnp.int32)
    sa, sb = _edges2(ei[0], CPWF, CPWS, N)
    da, db = _edges2(ei[1], CPWF, CPWS, N)

    xp = jnp.pad(x, ((0, NP - N), (0, 0)))
    return _gcn_pallas(xp, sa, sb, da, db, W2, b2, Wc, bc,
                       N, NP, NPT, CPWF, CPWS)


# speedup vs baseline: 1.3562x; 1.3562x over previous
import functools

import jax
import jax.numpy as jnp
from jax import lax
from jax.experimental import pallas as pl
from jax.experimental.pallas import tpu as pltpu
from jax.experimental.pallas import tpu_sc as plsc

NC = 2
NS = 16
NW = NC * NS
CHUNK = 128


def _round_up(v, m):
    return (v + m - 1) // m * m


@functools.partial(jax.jit, static_argnames=("n", "np_", "npt", "cpwf", "cpws"))
def _gcn_pallas(x, ei, W2, b2, Wc, bc, n, np_, npt, cpwf, cpws):
    _, D = x.shape
    N = n
    H = W2.shape[1]
    C = Wc.shape[1]
    E = ei.shape[1]
    NP = np_
    NPT = npt
    CPWF = cpwf
    CPWS = cpws
    LF = CPWF * CHUNK
    LS = CPWS * CHUNK
    E0 = NS * LF
    LTAIL = E - E0 - (NS - 1) * LS
    NWT = LTAIL // CHUNK
    assert 0 < LTAIL <= LS and LTAIL % CHUNK == 0 and NWT % 2 == 0

    mesh = plsc.VectorSubcoreMesh(core_axis_name="c", subcore_axis_name="s")

    def _stage_edges(ei_hbm, stage):
        c = lax.axis_index("c")
        s = lax.axis_index("s")

        @pl.when(c == 0)
        def _():
            pltpu.sync_copy(ei_hbm.at[:, pl.ds(s * LF, LF)], stage)

        @pl.when(jnp.logical_and(c != 0, s != NS - 1))
        def _():
            pltpu.sync_copy(ei_hbm.at[:, pl.ds(E0 + s * LS, LS)],
                            stage.at[:, pl.ds(0, LS)])

        @pl.when(jnp.logical_and(c != 0, s == NS - 1))
        def _():
            pltpu.sync_copy(ei_hbm.at[:, pl.ds(E0 + (NS - 1) * LS, LTAIL)],
                            stage.at[:, pl.ds(0, LTAIL)])

        return jnp.where(c == 0, CPWF,
                         jnp.where(s == NS - 1, NWT, CPWS))

    @functools.partial(
        pl.kernel,
        out_type=jax.ShapeDtypeStruct((NC, NP), jnp.float32),
        mesh=mesh,
        scratch_types=[
            pltpu.VMEM((2, LF), jnp.int32),
            pltpu.VMEM((CHUNK,), jnp.float32),
            pltpu.VMEM((NPT,), jnp.float32),
            pltpu.VMEM_SHARED((NP,), jnp.float32),
        ],
    )
    def deg_kernel(ei_hbm, out_hbm, idxs, ones_v, stage, deg_sh):
        c = lax.axis_index("c")
        s = lax.axis_index("s")
        nw = _stage_edges(ei_hbm, idxs)
        for i in range(CHUNK // 16):
            ones_v[pl.ds(i * 16, 16)] = jnp.full((16,), 1.0, jnp.float32)
        for i in range(NPT // 16):
            stage[pl.ds(i * 16, 16)] = jnp.zeros((16,), jnp.float32)
        pltpu.sync_copy(stage, deg_sh.at[pl.ds(s * NPT, NPT)])
        plsc.subcore_barrier()

        @pl.loop(0, nw)
        def _(j):
            pltpu.sync_copy(
                ones_v, deg_sh.at[idxs.at[1, pl.ds(j * CHUNK, CHUNK)]],
                add=True)

        plsc.subcore_barrier()
        pltpu.sync_copy(deg_sh.at[pl.ds(s * NPT, NPT)], stage)
        pltpu.sync_copy(stage, out_hbm.at[c, pl.ds(s * NPT, NPT)])

    degp = deg_kernel(ei)
    d0 = degp[0].reshape(NP, 1)
    d1 = degp[1].reshape(NP, 1)

    GB = 8
    RB = NP // GB
    HH = H // 2
    W2s = W2.reshape(D, 2, HH).transpose(1, 0, 2)

    def tc1_body(x_ref, w_ref, d0_ref, d1_ref, g_ref):
        dinv = lax.rsqrt(d0_ref[...] + d1_ref[...] + 1.0)
        xb = x_ref[...]
        g_ref[0] = jnp.dot(xb, w_ref[0],
                           preferred_element_type=jnp.float32) * dinv
        g_ref[1] = jnp.dot(xb, w_ref[1],
                           preferred_element_type=jnp.float32) * dinv

    g2 = pl.pallas_call(
        tc1_body,
        grid=(GB,),
        in_specs=[
            pl.BlockSpec((RB, D), lambda i: (i, 0)),
            pl.BlockSpec((2, D, HH), lambda i: (0, 0, 0)),
            pl.BlockSpec((RB, 1), lambda i: (i, 0)),
            pl.BlockSpec((RB, 1), lambda i: (i, 0)),
        ],
        out_specs=pl.BlockSpec((2, RB, HH), lambda i: (0, i, 0)),
        out_shape=jax.ShapeDtypeStruct((2, NP, HH), jnp.float32),
    )(x, W2s, d0, d1)

    @functools.partial(
        pl.kernel,
        out_type=jax.ShapeDtypeStruct((2, NC, NP, HH), jnp.float32),
        mesh=mesh,
        compiler_params=pltpu.CompilerParams(use_tc_tiling_on_sc=False),
        scratch_types=[
            pltpu.VMEM((2, LF), jnp.int32),
            pltpu.VMEM((CHUNK, HH), jnp.float32),
            pltpu.VMEM((CHUNK, HH), jnp.float32),
            pltpu.VMEM_SHARED((NP, HH), jnp.float32),
            pltpu.SemaphoreType.DMA,
            pltpu.SemaphoreType.DMA,
        ],
    )
    def agg_kernel(g2_hbm, ei_hbm, o_hbm,
                   eidx, r0, r1, agg_sh, sem0, sem1):
        c = lax.axis_index("c")
        s = lax.axis_index("s")
        nw = _stage_edges(ei_hbm, eidx)

        def sidx(j):
            return eidx.at[0, pl.ds(j * CHUNK, CHUNK)]

        def didx(j):
            return eidx.at[1, pl.ds(j * CHUNK, CHUNK)]

        for p in range(2):
            g_hbm = g2_hbm.at[p]
            out_hbm = o_hbm.at[p]
            @pl.loop(0, NPT // CHUNK)
            def _(i):
                base = s * NPT + i * CHUNK
                pltpu.sync_copy(g_hbm.at[pl.ds(base, CHUNK)], r0)
                pltpu.sync_copy(r0, agg_sh.at[pl.ds(base, CHUNK)])

            plsc.subcore_barrier()

            pltpu.async_copy(g_hbm.at[sidx(0)], r0, sem0)

            @pl.loop(0, nw, step=2)
            def _(j):
                pltpu.async_copy(g_hbm.at[sidx(j + 1)], r1, sem1)
                pltpu.make_async_copy(g_hbm.at[sidx(j)], r0, sem0).wait()
                pltpu.sync_copy(r0, agg_sh.at[didx(j)], add=True)

                @pl.when(j + 2 < nw)
                def _():
                    pltpu.async_copy(g_hbm.at[sidx(j + 2)], r0, sem0)

                pltpu.make_async_copy(g_hbm.at[sidx(j + 1)], r1,
                                      sem1).wait()
                pltpu.sync_copy(r1, agg_sh.at[didx(j + 1)], add=True)

            plsc.subcore_barrier()

            @pl.loop(0, NPT // CHUNK)
            def _(i):
                base = s * NPT + i * CHUNK
                pltpu.sync_copy(agg_sh.at[pl.ds(base, CHUNK)], r0)
                pltpu.sync_copy(r0, out_hbm.at[c, pl.ds(base, CHUNK)])

            plsc.subcore_barrier()

    o4 = agg_kernel(g2, ei)

    def tc2_body(a00, a01, a10, a11, ga_ref, gb_ref, d0_ref, d1_ref,
                 b2l, b2r, wcl, wcr, bc_ref, o_ref):
        dinv = lax.rsqrt(d0_ref[...] + d1_ref[...] + 1.0)
        pre_l = (a00[0, 0] + a01[0, 0] - ga_ref[0]) * dinv + b2l[...]
        pre_r = (a10[0, 0] + a11[0, 0] - gb_ref[0]) * dinv + b2r[...]
        pre_l = jnp.maximum(pre_l, 0.0)
        pre_r = jnp.maximum(pre_r, 0.0)
        acc = jnp.dot(pre_l, wcl[...], preferred_element_type=jnp.float32)
        acc += jnp.dot(pre_r, wcr[...], preferred_element_type=jnp.float32)
        o_ref[...] = acc + bc_ref[...]

    GB2 = 10
    RB2 = N // GB2

    def _ospec(p, cc):
        return pl.BlockSpec((1, 1, RB2, HH),
                            lambda i, p=p, cc=cc: (p, cc, i, 0))

    def _gspec(p):
        return pl.BlockSpec((1, RB2, HH), lambda i, p=p: (p, i, 0))

    out = pl.pallas_call(
        tc2_body,
        grid=(GB2,),
        in_specs=[
            _ospec(0, 0), _ospec(0, 1), _ospec(1, 0), _ospec(1, 1),
            _gspec(0), _gspec(1),
            pl.BlockSpec((RB2, 1), lambda i: (i, 0)),
            pl.BlockSpec((RB2, 1), lambda i: (i, 0)),
            pl.BlockSpec((1, HH), lambda i: (0, 0)),
            pl.BlockSpec((1, HH), lambda i: (0, 0)),
            pl.BlockSpec((HH, C), lambda i: (0, 0)),
            pl.BlockSpec((HH, C), lambda i: (0, 0)),
            pl.BlockSpec((1, C), lambda i: (0, 0)),
        ],
        out_specs=pl.BlockSpec((RB2, C), lambda i: (i, 0)),
        out_shape=jax.ShapeDtypeStruct((N, C), jnp.float32),
    )(o4, o4, o4, o4, g2, g2, d0, d1,
      b2[:HH].reshape(1, HH), b2[HH:].reshape(1, HH),
      Wc[:HH], Wc[HH:], bc.reshape(1, C))

    return out


FAST_FRAC = 0.72


def _split_counts(E):
    tch = -(-E // CHUNK)
    cpwf = _round_up(-(-int(tch * FAST_FRAC) // NS), 2)
    rem = max(E - NS * cpwf * CHUNK, 0)
    cpws = max(_round_up(-(-rem // (NS * CHUNK)), 2), 2)
    return cpwf, cpws


def kernel(x, edge_index, W2, b2, Wc, bc):
    N, D = x.shape
    H = W2.shape[1]
    C = Wc.shape[1]
    E = edge_index.shape[1]

    NP = _round_up(N + 1, NS * CHUNK)
    NPT = NP // NS
    CPWF, CPWS = _split_counts(E)

    ei = edge_index.astype(jnp.int32)
    xp = jnp.pad(x, ((0, NP - N), (0, 0)))
    return _gcn_pallas(xp, ei, W2, b2, Wc, bc, N, NP, NPT, CPWF, CPWS)


# speedup vs baseline: 1.6350x; 1.2055x over previous
import functools

import jax
import jax.numpy as jnp
from jax import lax
from jax.experimental import pallas as pl
from jax.experimental.pallas import tpu as pltpu
from jax.experimental.pallas import tpu_sc as plsc

NC = 2
NS = 16
NW = NC * NS
CHUNK = 128


def _round_up(v, m):
    return (v + m - 1) // m * m


@functools.partial(jax.jit, static_argnames=("n", "np_", "npt", "cpwf", "cpws"))
def _gcn_pallas(x, ei, W2, b2, Wc, bc, n, np_, npt, cpwf, cpws):
    _, D = x.shape
    N = n
    H = W2.shape[1]
    C = Wc.shape[1]
    E = ei.shape[1]
    NP = np_
    NPT = npt
    CPWF = cpwf
    CPWS = cpws
    LF = CPWF * CHUNK
    LS = CPWS * CHUNK
    E0 = NS * LF
    LTAIL = E - E0 - (NS - 1) * LS
    NWT = LTAIL // CHUNK
    assert 0 < LTAIL <= LS and LTAIL % CHUNK == 0 and NWT % 2 == 0

    mesh = plsc.VectorSubcoreMesh(core_axis_name="c", subcore_axis_name="s")

    def _stage_edges(ei_hbm, stage):
        c = lax.axis_index("c")
        s = lax.axis_index("s")

        @pl.when(c == 0)
        def _():
            pltpu.sync_copy(ei_hbm.at[:, pl.ds(s * LF, LF)], stage)

        @pl.when(jnp.logical_and(c != 0, s != NS - 1))
        def _():
            pltpu.sync_copy(ei_hbm.at[:, pl.ds(E0 + s * LS, LS)],
                            stage.at[:, pl.ds(0, LS)])

        @pl.when(jnp.logical_and(c != 0, s == NS - 1))
        def _():
            pltpu.sync_copy(ei_hbm.at[:, pl.ds(E0 + (NS - 1) * LS, LTAIL)],
                            stage.at[:, pl.ds(0, LTAIL)])

        return jnp.where(c == 0, CPWF,
                         jnp.where(s == NS - 1, NWT, CPWS))

    @functools.partial(
        pl.kernel,
        out_type=jax.ShapeDtypeStruct((NC, NP), jnp.float32),
        mesh=mesh,
        scratch_types=[
            pltpu.VMEM((2, LF), jnp.int32),
            pltpu.VMEM((CHUNK,), jnp.float32),
            pltpu.VMEM((NPT,), jnp.float32),
            pltpu.VMEM_SHARED((NP,), jnp.float32),
        ],
    )
    def deg_kernel(ei_hbm, out_hbm, idxs, ones_v, stage, deg_sh):
        c = lax.axis_index("c")
        s = lax.axis_index("s")
        nw = _stage_edges(ei_hbm, idxs)
        for i in range(CHUNK // 16):
            ones_v[pl.ds(i * 16, 16)] = jnp.full((16,), 1.0, jnp.float32)
        for i in range(NPT // 16):
            stage[pl.ds(i * 16, 16)] = jnp.zeros((16,), jnp.float32)
        pltpu.sync_copy(stage, deg_sh.at[pl.ds(s * NPT, NPT)])
        plsc.subcore_barrier()

        @pl.loop(0, nw)
        def _(j):
            pltpu.sync_copy(
                ones_v, deg_sh.at[idxs.at[1, pl.ds(j * CHUNK, CHUNK)]],
                add=True)

        plsc.subcore_barrier()
        pltpu.sync_copy(deg_sh.at[pl.ds(s * NPT, NPT)], stage)
        pltpu.sync_copy(stage, out_hbm.at[c, pl.ds(s * NPT, NPT)])

    degp = deg_kernel(ei)
    d0 = degp[0].reshape(NP, 1)
    d1 = degp[1].reshape(NP, 1)

    GB = 8
    RB = NP // GB
    HH = H // 2
    W2s = W2.reshape(D, 2, HH).transpose(1, 0, 2)

    def tc1_body(x_ref, w_ref, d0_ref, d1_ref, g_ref):
        dinv = lax.rsqrt(d0_ref[...] + d1_ref[...] + 1.0)
        xb = x_ref[...]
        g_ref[0] = jnp.dot(xb, w_ref[0],
                           preferred_element_type=jnp.float32) * dinv
        g_ref[1] = jnp.dot(xb, w_ref[1],
                           preferred_element_type=jnp.float32) * dinv

    g2 = pl.pallas_call(
        tc1_body,
        grid=(GB,),
        in_specs=[
            pl.BlockSpec((RB, D), lambda i: (i, 0)),
            pl.BlockSpec((2, D, HH), lambda i: (0, 0, 0)),
            pl.BlockSpec((RB, 1), lambda i: (i, 0)),
            pl.BlockSpec((RB, 1), lambda i: (i, 0)),
        ],
        out_specs=pl.BlockSpec((2, RB, HH), lambda i: (0, i, 0)),
        out_shape=jax.ShapeDtypeStruct((2, NP, HH), jnp.float32),
    )(x, W2s, d0, d1)

    @functools.partial(
        pl.kernel,
        out_type=jax.ShapeDtypeStruct((2, NC, NP, HH), jnp.float32),
        mesh=mesh,
        compiler_params=pltpu.CompilerParams(use_tc_tiling_on_sc=False),
        scratch_types=[
            pltpu.VMEM((2, LF), jnp.int32),
            pltpu.VMEM((CHUNK, HH), jnp.float32),
            pltpu.VMEM((CHUNK, HH), jnp.float32),
            pltpu.VMEM_SHARED((NP, HH), jnp.float32),
            pltpu.SemaphoreType.DMA,
            pltpu.SemaphoreType.DMA,
        ],
    )
    def agg_kernel(g2_hbm, ei_hbm, o_hbm,
                   eidx, r0, r1, agg_sh, sem0, sem1):
        c = lax.axis_index("c")
        s = lax.axis_index("s")
        nw = _stage_edges(ei_hbm, eidx)

        def sidx(j):
            return eidx.at[0, pl.ds(j * CHUNK, CHUNK)]

        def didx(j):
            return eidx.at[1, pl.ds(j * CHUNK, CHUNK)]

        for p in range(2):
            g_hbm = g2_hbm.at[p]
            out_hbm = o_hbm.at[p]
            @pl.loop(0, NPT // CHUNK)
            def _(i):
                base = s * NPT + i * CHUNK
                pltpu.sync_copy(g_hbm.at[pl.ds(base, CHUNK)], r0)
                pltpu.sync_copy(r0, agg_sh.at[pl.ds(base, CHUNK)])

            plsc.subcore_barrier()

            pltpu.async_copy(g_hbm.at[sidx(0)], r0, sem0)

            @pl.loop(0, nw, step=2)
            def _(j):
                pltpu.async_copy(g_hbm.at[sidx(j + 1)], r1, sem1)
                pltpu.make_async_copy(g_hbm.at[sidx(j)], r0, sem0).wait()
                pltpu.sync_copy(r0, agg_sh.at[didx(j)], add=True)

                @pl.when(j + 2 < nw)
                def _():
                    pltpu.async_copy(g_hbm.at[sidx(j + 2)], r0, sem0)

                pltpu.make_async_copy(g_hbm.at[sidx(j + 1)], r1,
                                      sem1).wait()
                pltpu.sync_copy(r1, agg_sh.at[didx(j + 1)], add=True)

            plsc.subcore_barrier()

            @pl.loop(0, NPT // CHUNK)
            def _(i):
                base = s * NPT + i * CHUNK
                pltpu.sync_copy(agg_sh.at[pl.ds(base, CHUNK)], r0)
                pltpu.sync_copy(r0, out_hbm.at[c, pl.ds(base, CHUNK)])

            plsc.subcore_barrier()

    o4 = agg_kernel(g2, ei)

    def tc2_body(a00, a01, a10, a11, ga_ref, gb_ref, d0_ref, d1_ref,
                 b2l, b2r, wcl, wcr, bc_ref, o_ref):
        dinv = lax.rsqrt(d0_ref[...] + d1_ref[...] + 1.0)
        pre_l = (a00[0, 0] + a01[0, 0] - ga_ref[0]) * dinv + b2l[...]
        pre_r = (a10[0, 0] + a11[0, 0] - gb_ref[0]) * dinv + b2r[...]
        pre_l = jnp.maximum(pre_l, 0.0)
        pre_r = jnp.maximum(pre_r, 0.0)
        acc = jnp.dot(pre_l, wcl[...], preferred_element_type=jnp.float32)
        acc += jnp.dot(pre_r, wcr[...], preferred_element_type=jnp.float32)
        o_ref[...] = acc + bc_ref[...]

    GB2 = 10
    RB2 = N // GB2

    def _ospec(p, cc):
        return pl.BlockSpec((1, 1, RB2, HH),
                            lambda i, p=p, cc=cc: (p, cc, i, 0))

    def _gspec(p):
        return pl.BlockSpec((1, RB2, HH), lambda i, p=p: (p, i, 0))

    out = pl.pallas_call(
        tc2_body,
        grid=(GB2,),
        in_specs=[
            _ospec(0, 0), _ospec(0, 1), _ospec(1, 0), _ospec(1, 1),
            _gspec(0), _gspec(1),
            pl.BlockSpec((RB2, 1), lambda i: (i, 0)),
            pl.BlockSpec((RB2, 1), lambda i: (i, 0)),
            pl.BlockSpec((1, HH), lambda i: (0, 0)),
            pl.BlockSpec((1, HH), lambda i: (0, 0)),
            pl.BlockSpec((HH, C), lambda i: (0, 0)),
            pl.BlockSpec((HH, C), lambda i: (0, 0)),
            pl.BlockSpec((1, C), lambda i: (0, 0)),
        ],
        out_specs=pl.BlockSpec((RB2, C), lambda i: (i, 0)),
        out_shape=jax.ShapeDtypeStruct((N, C), jnp.float32),
    )(o4, o4, o4, o4, g2, g2, d0, d1,
      b2[:HH].reshape(1, HH), b2[HH:].reshape(1, HH),
      Wc[:HH], Wc[HH:], bc.reshape(1, C))

    return out


FAST_FRAC = 0.5


def _split_counts(E):
    tch = -(-E // CHUNK)
    cpwf = _round_up(-(-int(tch * FAST_FRAC) // NS), 2)
    rem = max(E - NS * cpwf * CHUNK, 0)
    cpws = max(_round_up(-(-rem // (NS * CHUNK)), 2), 2)
    return cpwf, cpws


def kernel(x, edge_index, W2, b2, Wc, bc):
    N, D = x.shape
    H = W2.shape[1]
    C = Wc.shape[1]
    E = edge_index.shape[1]

    NP = _round_up(N + 1, NS * CHUNK)
    NPT = NP // NS
    CPWF, CPWS = _split_counts(E)

    ei = edge_index.astype(jnp.int32)
    xp = jnp.pad(x, ((0, NP - N), (0, 0)))
    return _gcn_pallas(xp, ei, W2, b2, Wc, bc, N, NP, NPT, CPWF, CPWS)
